# baseline (speedup 1.0000x reference)
import functools

import jax
import jax.numpy as jnp
from jax import lax
from jax.experimental import pallas as pl
from jax.experimental.pallas import tpu as pltpu
from jax.experimental.pallas import tpu_sc as plsc

CLS = 100000
FEAT = 64
BATCH = 16384

NC = 2
NS = 16
NW = NC * NS
RPW = BATCH // NW
CP = 100352
ZPW = CP // NS
NIDX = RPW // 128

_MESH = plsc.VectorSubcoreMesh(
    core_axis_name="c", subcore_axis_name="s", num_cores=NC, num_subcores=NS
)


def _rsqrt(x):
    i = lax.bitcast_convert_type(x, jnp.int32)
    i = 0x5F3759DF - lax.shift_right_arithmetic(i, 1)
    y = lax.bitcast_convert_type(i, jnp.float32)
    for _ in range(3):
        y = y * (1.5 - 0.5 * x * y * y)
    return y


def _stage1_body(xs_hbm, ys_hbm, center_hbm, out_hbm,
                 xs_v, cr_v, ys_v, idx_v, dist_v, ones_v, zeros_v,
                 ps_v, pp_v, pt_v, cnt_sh, dsum_sh, sem_g, sem_x):
    cid = lax.axis_index("c")
    sid = lax.axis_index("s")
    wid = cid * NS + sid
    base = wid * RPW

    xs_cp = pltpu.async_copy(xs_hbm.at[pl.ds(base, RPW)], xs_v, sem_x)
    pltpu.sync_copy(ys_hbm.at[pl.ds(base, RPW)], ys_v)

    zero16 = jnp.zeros((16,), jnp.float32)
    one16 = jnp.ones((16,), jnp.float32)

    def _fill_zeros(i, carry):
        zeros_v[pl.ds(i * 16, 16)] = zero16
        return carry

    lax.fori_loop(0, ZPW // 16, _fill_zeros, 0)

    def _fill_ones(i, carry):
        ones_v[pl.ds(i * 16, 16)] = one16
        return carry

    lax.fori_loop(0, RPW // 16, _fill_ones, 0)

    zslice = pl.ds(sid * ZPW, ZPW)
    pltpu.sync_copy(zeros_v, cnt_sh.at[zslice])
    pltpu.sync_copy(zeros_v, dsum_sh.at[zslice])

    for r in range(NIDX):
        for c8 in range(8):
            off = (r * 8 + c8) * 16
            idx_v[r, pl.ds(c8 * 16, 16)] = ys_v[pl.ds(off, 16)].astype(jnp.int32)

    descs = [
        pltpu.async_copy(center_hbm.at[idx_v.at[j]],
                         cr_v.at[pl.ds(j * 128, 128)], sem_g)
        for j in range(NIDX)
    ]
    for d in descs:
        d.wait()
    xs_cp.wait()

    def _row(r, carry):
        xrow = xs_v.at[r]
        crow = cr_v.at[r]
        s = zero16
        p = zero16
        t = zero16
        for k in range(FEAT // 16):
            xv = xrow[pl.ds(k * 16, 16)]
            cv = crow[pl.ds(k * 16, 16)]
            s = s + xv * xv
            p = p + xv * cv
            t = t + cv * cv
        ps_v[pl.ds(r * 16, 16)] = s
        pp_v[pl.ds(r * 16, 16)] = p
        pt_v[pl.ds(r * 16, 16)] = t
        return carry

    lax.fori_loop(0, RPW, _row, 0)

    iota16 = lax.broadcasted_iota(jnp.int32, (16,), 0)
    iota_r16 = iota16 * 16

    def _group(g, carry):
        lin0 = g * 256 + iota_r16
        s = zero16
        p = zero16
        t = zero16
        for l in range(16):
            s = s + plsc.load_gather(ps_v, [lin0 + l])
            p = p + plsc.load_gather(pp_v, [lin0 + l])
            t = t + plsc.load_gather(pt_v, [lin0 + l])
        inv = _rsqrt(jnp.maximum(s, 1e-24))
        q = jnp.maximum(s * inv * inv - 2.0 * inv * p + t, 0.0)
        dist = q * _rsqrt(jnp.maximum(q, 1e-36))
        dist_v[pl.ds(g * 16, 16)] = dist
        return carry

    lax.fori_loop(0, RPW // 16, _group, 0)

    plsc.subcore_barrier()
    for j in range(NIDX):
        sl = pl.ds(j * 128, 128)
        pltpu.sync_copy(ones_v.at[sl], cnt_sh.at[idx_v.at[j]], add=True)
        pltpu.sync_copy(dist_v.at[sl], dsum_sh.at[idx_v.at[j]], add=True)
    plsc.subcore_barrier()

    pltpu.sync_copy(cnt_sh.at[zslice], out_hbm.at[cid, 0, zslice])
    pltpu.sync_copy(dsum_sh.at[zslice], out_hbm.at[cid, 1, zslice])


_stage1 = functools.partial(
    pl.kernel,
    out_type=jax.ShapeDtypeStruct((NC, 2, CP), jnp.float32),
    mesh=_MESH,
    scratch_types=[
        pltpu.VMEM((RPW, FEAT), jnp.float32),
        pltpu.VMEM((RPW, FEAT), jnp.float32),
        pltpu.VMEM((RPW,), jnp.float32),
        pltpu.VMEM((NIDX, 128), jnp.int32),
        pltpu.VMEM((RPW,), jnp.float32),
        pltpu.VMEM((RPW,), jnp.float32),
        pltpu.VMEM((ZPW,), jnp.float32),
        pltpu.VMEM((RPW * 16,), jnp.float32),
        pltpu.VMEM((RPW * 16,), jnp.float32),
        pltpu.VMEM((RPW * 16,), jnp.float32),
        pltpu.VMEM_SHARED((CP,), jnp.float32),
        pltpu.VMEM_SHARED((CP,), jnp.float32),
        pltpu.SemaphoreType.DMA,
        pltpu.SemaphoreType.DMA,
    ],
    compiler_params=pltpu.CompilerParams(
        needs_layout_passes=False, use_tc_tiling_on_sc=False),
)(_stage1_body)


def _combine_body(tab_ref, out_ref):
    cnt = tab_ref[0] + tab_ref[2]
    tot = tab_ref[1] + tab_ref[3]
    out_ref[...] = jnp.sum(tot / jnp.maximum(cnt, 1.0)).reshape(1, 1)


def kernel(xs, ys, center):
    tab = _stage1(xs, ys, center)
    loss = pl.pallas_call(
        _combine_body,
        out_shape=jax.ShapeDtypeStruct((1, 1), jnp.float32),
    )(tab.reshape(4, CP // 128, 128))
    return loss[0, 0]


# speedup vs baseline: 1.1033x; 1.1033x over previous
import functools

import jax
import jax.numpy as jnp
from jax import lax
from jax.experimental import pallas as pl
from jax.experimental.pallas import tpu as pltpu
from jax.experimental.pallas import tpu_sc as plsc

CLS = 100000
FEAT = 64
BATCH = 16384

NC = 2
NS = 16
NW = NC * NS
RPW = BATCH // NW
RH = RPW // 2
CP = 100352
ZPW = CP // NS
GCH = 16

_MESH = plsc.VectorSubcoreMesh(
    core_axis_name="c", subcore_axis_name="s", num_cores=NC, num_subcores=NS
)


def _rsqrt(x):
    i = lax.bitcast_convert_type(x, jnp.int32)
    i = 0x5F3759DF - lax.shift_right_arithmetic(i, 1)
    y = lax.bitcast_convert_type(i, jnp.float32)
    for _ in range(3):
        y = y * (1.5 - 0.5 * x * y * y)
    return y


def _stage_a_body(xs_hbm, ys_hbm, center_hbm, dist_hbm, idx_hbm,
                  xs_v, cr_v, ys_v, idx_v, dist_v, ps_v, pp_v, pt_v,
                  sem_g, sem_x):
    cid = lax.axis_index("c")
    sid = lax.axis_index("s")
    wid = cid * NS + sid
    base = wid * RPW

    pltpu.sync_copy(ys_hbm.at[pl.ds(wid * 4, 4)], ys_v)
    for r in range(4):
        for c8 in range(8):
            sl = pl.ds(c8 * 16, 16)
            idx_v[r, sl] = ys_v[r, sl].astype(jnp.int32)

    zero16 = jnp.zeros((16,), jnp.float32)
    iota16 = lax.broadcasted_iota(jnp.int32, (16,), 0)

    for h in range(2):
        xs_cp = pltpu.async_copy(
            xs_hbm.at[pl.ds(base + h * RH, RH)], xs_v, sem_x)

        def _gchunk(c, carry):
            flat = h * RH + c * GCH
            rowv = idx_v[flat // 128, pl.ds((flat % 128) * 1, GCH)]
            for k in range(GCH):
                pltpu.async_copy(
                    center_hbm.at[rowv[k]], cr_v.at[c * GCH + k], sem_g)

            @pl.when(c > 0)
            def _():
                pltpu.make_async_copy(
                    center_hbm.at[pl.ds(0, GCH)],
                    cr_v.at[pl.ds(0, GCH)], sem_g).wait()

            return carry

        lax.fori_loop(0, RH // GCH, _gchunk, 0, unroll=False)
        pltpu.make_async_copy(
            center_hbm.at[pl.ds(0, GCH)], cr_v.at[pl.ds(0, GCH)],
            sem_g).wait()
        xs_cp.wait()

        def _row(r, carry):
            s = zero16
            p = zero16
            t = zero16
            for k in range(FEAT // 16):
                sl = pl.ds(k * 16, 16)
                xv = xs_v[r, sl]
                cv = cr_v[r, sl]
                s = s + xv * xv
                p = p + xv * cv
                t = t + cv * cv
            prow = r // 8
            psl = pl.ds((r % 8) * 16, 16)
            ps_v[prow, psl] = s
            pp_v[prow, psl] = p
            pt_v[prow, psl] = t
            return carry

        lax.fori_loop(0, RH, _row, 0, unroll=False)

        def _group(g, carry):
            lin0 = g * 256 + iota16 * 16
            s = zero16
            p = zero16
            t = zero16
            for l in range(16):
                lin = lin0 + l
                ri = lax.shift_right_logical(lin, 7)
                ci = lax.bitwise_and(lin, 127)
                s = s + plsc.load_gather(ps_v, [ri, ci])
                p = p + plsc.load_gather(pp_v, [ri, ci])
                t = t + plsc.load_gather(pt_v, [ri, ci])
            inv = _rsqrt(jnp.maximum(s, 1e-24))
            q = jnp.maximum(s * inv * inv - 2.0 * inv * p + t, 0.0)
            dist = q * _rsqrt(jnp.maximum(q, 1e-36))
            flat = h * RH + g * 16
            dist_v[flat // 128, pl.ds((flat % 128) * 1, 16)] = dist
            return carry

        lax.fori_loop(0, RH // 16, _group, 0, unroll=False)

    pltpu.sync_copy(dist_v, dist_hbm.at[pl.ds(wid * 4, 4)])
    pltpu.sync_copy(idx_v, idx_hbm.at[pl.ds(wid * 4, 4)])


_stage_a = functools.partial(
    pl.kernel,
    out_type=(
        jax.ShapeDtypeStruct((128, 128), jnp.float32),
        jax.ShapeDtypeStruct((128, 128), jnp.int32),
    ),
    mesh=_MESH,
    scratch_types=[
        pltpu.VMEM((RH, FEAT), jnp.float32),
        pltpu.VMEM((RH, FEAT), jnp.float32),
        pltpu.VMEM((4, 128), jnp.float32),
        pltpu.VMEM((4, 128), jnp.int32),
        pltpu.VMEM((4, 128), jnp.float32),
        pltpu.VMEM((RH // 8, 128), jnp.float32),
        pltpu.VMEM((RH // 8, 128), jnp.float32),
        pltpu.VMEM((RH // 8, 128), jnp.float32),
        pltpu.SemaphoreType.DMA,
        pltpu.SemaphoreType.DMA,
    ],
    compiler_params=pltpu.CompilerParams(
        needs_layout_passes=False, use_tc_tiling_on_sc=True),
)(_stage_a_body)


def _stage_b_body(dist_hbm, idx_hbm, out_hbm,
                  dist_v, idx_v, ones_v, zeros_v, cnt_sh, dsum_sh):
    cid = lax.axis_index("c")
    sid = lax.axis_index("s")
    wid = cid * NS + sid

    pltpu.sync_copy(dist_hbm.at[pl.ds(wid * 4, 4)], dist_v)
    pltpu.sync_copy(idx_hbm.at[pl.ds(wid * 4, 4)], idx_v)

    zero16 = jnp.zeros((16,), jnp.float32)
    one16 = jnp.ones((16,), jnp.float32)

    def _fill_zeros(i, carry):
        zeros_v[pl.ds(i * 16, 16)] = zero16
        return carry

    lax.fori_loop(0, ZPW // 16, _fill_zeros, 0)

    def _fill_ones(i, carry):
        ones_v[pl.ds(i * 16, 16)] = one16
        return carry

    lax.fori_loop(0, 128 // 16, _fill_ones, 0)

    zslice = pl.ds(sid * ZPW, ZPW)
    pltpu.sync_copy(zeros_v, cnt_sh.at[zslice])
    pltpu.sync_copy(zeros_v, dsum_sh.at[zslice])

    plsc.subcore_barrier()
    for j in range(4):
        pltpu.sync_copy(ones_v, cnt_sh.at[idx_v.at[j]], add=True)
        pltpu.sync_copy(dist_v.at[j], dsum_sh.at[idx_v.at[j]], add=True)
    plsc.subcore_barrier()

    obase = cid * (2 * CP) + sid * ZPW
    pltpu.sync_copy(cnt_sh.at[zslice], out_hbm.at[pl.ds(obase, ZPW)])
    pltpu.sync_copy(dsum_sh.at[zslice], out_hbm.at[pl.ds(obase + CP, ZPW)])


_stage_b = functools.partial(
    pl.kernel,
    out_type=jax.ShapeDtypeStruct((2 * NC * CP,), jnp.float32),
    mesh=_MESH,
    scratch_types=[
        pltpu.VMEM((4, 128), jnp.float32),
        pltpu.VMEM((4, 128), jnp.int32),
        pltpu.VMEM((128,), jnp.float32),
        pltpu.VMEM((ZPW,), jnp.float32),
        pltpu.VMEM_SHARED((CP,), jnp.float32),
        pltpu.VMEM_SHARED((CP,), jnp.float32),
    ],
    compiler_params=pltpu.CompilerParams(
        needs_layout_passes=False, use_tc_tiling_on_sc=False),
)(_stage_b_body)


def _combine_body(tab_ref, out_ref):
    cnt = tab_ref[0] + tab_ref[2]
    tot = tab_ref[1] + tab_ref[3]
    out_ref[...] = jnp.sum(tot / jnp.maximum(cnt, 1.0)).reshape(1, 1)


def kernel(xs, ys, center):
    dist2d, idx2d = _stage_a(xs, ys.reshape(128, 128), center)
    tab = _stage_b(dist2d, idx2d)
    loss = pl.pallas_call(
        _combine_body,
        out_shape=jax.ShapeDtypeStruct((1, 1), jnp.float32),
    )(tab.reshape(4, CP // 128, 128))
    return loss[0, 0]


# speedup vs baseline: 1.3371x; 1.2119x over previous
import functools

import jax
import jax.numpy as jnp
from jax import lax
from jax.experimental import pallas as pl
from jax.experimental.pallas import tpu as pltpu
from jax.experimental.pallas import tpu_sc as plsc
from jax.experimental.layout import Layout, with_layout_constraint

CLS = 100000
FEAT = 64
BATCH = 16384

NC = 2
NS = 16
NW = NC * NS
RPW = BATCH // NW
CP = 100352
ZPW = CP // NS
NIDX = RPW // 128

_MESH = plsc.VectorSubcoreMesh(
    core_axis_name="c", subcore_axis_name="s", num_cores=NC, num_subcores=NS
)


def _rsqrt(x):
    i = lax.bitcast_convert_type(x, jnp.int32)
    i = 0x5F3759DF - lax.shift_right_arithmetic(i, 1)
    y = lax.bitcast_convert_type(i, jnp.float32)
    for _ in range(3):
        y = y * (1.5 - 0.5 * x * y * y)
    return y


def _stage1_body(xs_hbm, ys_hbm, center_hbm, out_hbm,
                 xs_v, cr_v, ys_v, idx_v, dist_v, ones_v, zeros_v,
                 ps_v, pp_v, pt_v, cnt_sh, dsum_sh, sem_g, sem_x):
    cid = lax.axis_index("c")
    sid = lax.axis_index("s")
    wid = cid * NS + sid
    base = wid * RPW

    xs_cp = pltpu.async_copy(xs_hbm.at[pl.ds(base, RPW)], xs_v, sem_x)
    pltpu.sync_copy(ys_hbm.at[pl.ds(base, RPW)], ys_v)

    zero16 = jnp.zeros((16,), jnp.float32)
    one16 = jnp.ones((16,), jnp.float32)

    def _fill_zeros(i, carry):
        zeros_v[pl.ds(i * 16, 16)] = zero16
        return carry

    lax.fori_loop(0, ZPW // 16, _fill_zeros, 0)

    def _fill_ones(i, carry):
        ones_v[pl.ds(i * 16, 16)] = one16
        return carry

    lax.fori_loop(0, RPW // 16, _fill_ones, 0)

    zslice = pl.ds(sid * ZPW, ZPW)
    pltpu.sync_copy(zeros_v, cnt_sh.at[zslice])
    pltpu.sync_copy(zeros_v, dsum_sh.at[zslice])

    for r in range(NIDX):
        for c8 in range(8):
            off = (r * 8 + c8) * 16
            idx_v[r, pl.ds(c8 * 16, 16)] = ys_v[pl.ds(off, 16)].astype(jnp.int32)

    descs = [
        pltpu.async_copy(center_hbm.at[idx_v.at[j]],
                         cr_v.at[pl.ds(j * 128, 128)], sem_g)
        for j in range(NIDX)
    ]
    for d in descs:
        d.wait()
    xs_cp.wait()

    def _row(r, carry):
        xrow = xs_v.at[r]
        crow = cr_v.at[r]
        s = zero16
        p = zero16
        t = zero16
        for k in range(FEAT // 16):
            xv = xrow[pl.ds(k * 16, 16)]
            cv = crow[pl.ds(k * 16, 16)]
            s = s + xv * xv
            p = p + xv * cv
            t = t + cv * cv
        ps_v[pl.ds(r * 16, 16)] = s
        pp_v[pl.ds(r * 16, 16)] = p
        pt_v[pl.ds(r * 16, 16)] = t
        return carry

    lax.fori_loop(0, RPW, _row, 0)

    iota16 = lax.broadcasted_iota(jnp.int32, (16,), 0)
    iota_r16 = iota16 * 16

    def _group(g, carry):
        lin0 = g * 256 + iota_r16
        s = zero16
        p = zero16
        t = zero16
        for l in range(16):
            s = s + plsc.load_gather(ps_v, [lin0 + l])
            p = p + plsc.load_gather(pp_v, [lin0 + l])
            t = t + plsc.load_gather(pt_v, [lin0 + l])
        inv = _rsqrt(jnp.maximum(s, 1e-24))
        q = jnp.maximum(s * inv * inv - 2.0 * inv * p + t, 0.0)
        dist = q * _rsqrt(jnp.maximum(q, 1e-36))
        dist_v[pl.ds(g * 16, 16)] = dist
        return carry

    lax.fori_loop(0, RPW // 16, _group, 0)

    plsc.subcore_barrier()
    for j in range(NIDX):
        sl = pl.ds(j * 128, 128)
        pltpu.sync_copy(ones_v.at[sl], cnt_sh.at[idx_v.at[j]], add=True)
        pltpu.sync_copy(dist_v.at[sl], dsum_sh.at[idx_v.at[j]], add=True)
    plsc.subcore_barrier()

    obase = cid * (2 * CP) + sid * ZPW
    pltpu.sync_copy(cnt_sh.at[zslice], out_hbm.at[pl.ds(obase, ZPW)])
    pltpu.sync_copy(dsum_sh.at[zslice], out_hbm.at[pl.ds(obase + CP, ZPW)])


_stage1 = functools.partial(
    pl.kernel,
    out_type=jax.ShapeDtypeStruct((2 * NC * CP,), jnp.float32),
    mesh=_MESH,
    scratch_types=[
        pltpu.VMEM((RPW, FEAT), jnp.float32),
        pltpu.VMEM((RPW, FEAT), jnp.float32),
        pltpu.VMEM((RPW,), jnp.float32),
        pltpu.VMEM((NIDX, 128), jnp.int32),
        pltpu.VMEM((RPW,), jnp.float32),
        pltpu.VMEM((RPW,), jnp.float32),
        pltpu.VMEM((ZPW,), jnp.float32),
        pltpu.VMEM((RPW * 16,), jnp.float32),
        pltpu.VMEM((RPW * 16,), jnp.float32),
        pltpu.VMEM((RPW * 16,), jnp.float32),
        pltpu.VMEM_SHARED((CP,), jnp.float32),
        pltpu.VMEM_SHARED((CP,), jnp.float32),
        pltpu.SemaphoreType.DMA,
        pltpu.SemaphoreType.DMA,
    ],
    compiler_params=pltpu.CompilerParams(
        needs_layout_passes=False, use_tc_tiling_on_sc=False),
)(_stage1_body)


def _combine_body(tab_ref, out_ref):
    cnt = tab_ref[0] + tab_ref[2]
    tot = tab_ref[1] + tab_ref[3]
    out_ref[...] = jnp.sum(tot / jnp.maximum(cnt, 1.0)).reshape(1, 1)


_SC_LINEAR_2D = Layout(major_to_minor=(0, 1), tiling=((16,),))


def kernel(xs, ys, center):
    xs_l = with_layout_constraint(xs, _SC_LINEAR_2D)
    center_l = with_layout_constraint(center, _SC_LINEAR_2D)
    tab = _stage1(xs_l, ys, center_l)
    loss = pl.pallas_call(
        _combine_body,
        out_shape=jax.ShapeDtypeStruct((1, 1), jnp.float32),
    )(tab.reshape(4, CP // 128, 128))
    return loss[0, 0]
